# Initial kernel scaffold; baseline (speedup 1.0000x reference)
import jax
import jax.numpy as jnp
from jax.experimental import pallas as pl


def kernel(features, features_text, gt, phn_id):
    raise NotImplementedError("write your pallas kernel here")


# speedup vs baseline: 4.4584x; 4.4584x over previous
import jax
import jax.numpy as jnp
from jax import lax
from jax.experimental import pallas as pl

_LAMBDA_D_PHN = 0.1
_LAMBDA_T_PHN = 1.0
_LAMBDA_CLAP_T2A = 0.5
_MARGIN = 0.2
_P = 40
_F = 256


def _keep_kernel(gt_ref, phn_ref, keep_ref):
    gt = gt_ref[...]
    phn = phn_ref[...]
    iota = lax.broadcasted_iota(jnp.int32, (1, _P), 1)
    oh = phn == iota
    mn = gt > 0
    mh = gt == 2
    cn = jnp.sum(jnp.where(oh & mn, 1.0, 0.0), axis=0, keepdims=True)
    ch = jnp.sum(jnp.where(oh & mh, 1.0, 0.0), axis=0, keepdims=True)
    present_norm = cn > 0.0
    present_high = ch > 0.0
    skip = present_norm & (~present_high)
    any_skip = jnp.sum(jnp.where(skip, 1.0, 0.0)) > 0.0
    has_nonskip = jnp.sum(jnp.where(present_norm & (~skip), 1.0, 0.0)) > 0.0
    keep_if_skip = ~((iota == 1) | ((iota == 0) & has_nonskip))
    keep_if_skip_f = jnp.where(keep_if_skip, 1.0, 0.0)
    keep_ref[...] = jnp.where(any_skip, keep_if_skip_f, 1.0)


def _sums_kernel(gt_ref, phn_ref, keep_ref, f_ref, ft_ref,
                 sums_ref, sums_t_ref, cnt_ref):
    i = pl.program_id(0)

    @pl.when(i == 0)
    def _():
        sums_ref[...] = jnp.zeros_like(sums_ref)
        sums_t_ref[...] = jnp.zeros_like(sums_t_ref)
        cnt_ref[...] = jnp.zeros_like(cnt_ref)

    phn = phn_ref[...]
    gt = gt_ref[...]
    iota = lax.broadcasted_iota(jnp.int32, (1, _P), 1)
    oh = (phn == iota) & (gt > 0)
    ohf = jnp.where(oh, 1.0, 0.0) * keep_ref[...]
    dn = (((0,), (0,)), ((), ()))
    sums_ref[...] += lax.dot_general(ohf, f_ref[...], dn,
                                     preferred_element_type=jnp.float32)
    sums_t_ref[...] += lax.dot_general(ohf, ft_ref[...], dn,
                                       preferred_element_type=jnp.float32)
    cnt_ref[...] += jnp.sum(ohf, axis=0, keepdims=True)


def _centers_kernel(sums_ref, sums_t_ref, cnt_ref,
                    p_ref, pn2_ref, clap_ref, ent_ref):
    r = lax.broadcasted_iota(jnp.int32, (_P, _P), 0)
    c = lax.broadcasted_iota(jnp.int32, (_P, _P), 1)
    eye = jnp.where(r == c, 1.0, 0.0)

    def row_to_col(u):
        return lax.dot_general(eye, u, (((1,), (1,)), ((), ())),
                               preferred_element_type=jnp.float32)

    def col_to_row(v):
        return lax.dot_general(v, eye, (((0,), (0,)), ((), ())),
                               preferred_element_type=jnp.float32)

    cnt = cnt_ref[...]
    present = cnt > 0.0
    counts = jnp.where(present, cnt, 1.0)
    counts_c = row_to_col(counts)
    n_u = jnp.sum(jnp.where(present, 1.0, 0.0))

    def norm_rows(x):
        n = jnp.sqrt(jnp.sum(x * x, axis=1, keepdims=True))
        return x / jnp.maximum(n, 1e-12)

    center = norm_rows(sums_ref[...] / counts_c)
    center_t = norm_rows(sums_t_ref[...] / counts_c)

    dn = (((1,), (1,)), ((), ()))
    logits = lax.dot_general(center, center_t, dn,
                             preferred_element_type=jnp.float32)
    neg_inf = jnp.float32(-jnp.inf)
    logits = jnp.where(present, logits, neg_inf)
    m = jnp.max(logits, axis=1, keepdims=True)
    lse = jnp.log(jnp.sum(jnp.exp(logits - m), axis=1, keepdims=True)) + m
    cos = logits - lse

    diag = jnp.sum(jnp.where(r == c, cos, 0.0), axis=0, keepdims=True)
    loss_a = -jnp.sum(jnp.where(present, diag, 0.0)) / n_u
    clap_ref[...] = jnp.full((1, 1), 0.0) + loss_a

    p = norm_rows(center)
    xx = jnp.sum(p * p, axis=1, keepdims=True)
    xx_r = col_to_row(xx)
    gram = lax.dot_general(p, p, dn, preferred_element_type=jnp.float32)
    d2 = jnp.maximum(xx + xx_r - 2.0 * gram, 1e-12)
    dist = jnp.sqrt(d2)
    present_c = row_to_col(jnp.where(present, 1.0, 0.0)) > 0.0
    pair_mask = present_c & present & (c > r)
    n_pairs = n_u * (n_u - 1.0) / 2.0
    ent_ref[...] = jnp.full((1, 1), 0.0) + (
        jnp.sum(jnp.where(pair_mask, dist, 0.0)) / n_pairs)

    p_ref[...] = p
    pn2_ref[...] = xx_r


def _tight_kernel(gt_ref, phn_ref, keep_ref, f_ref, p_ref, pn2_ref, ent_ref,
                  clap_in_ref, ts_ref, tc_ref, loss_ref, clap_out_ref):
    i = pl.program_id(0)
    n = pl.num_programs(0)

    @pl.when(i == 0)
    def _():
        ts_ref[...] = jnp.zeros_like(ts_ref)
        tc_ref[...] = jnp.zeros_like(tc_ref)

    phn = phn_ref[...]
    gt = gt_ref[...]
    f = f_ref[...]
    iota = lax.broadcasted_iota(jnp.int32, (1, _P), 1)
    oh = phn == iota

    sq = jnp.sum(f * f, axis=1, keepdims=True)
    dn = (((1,), (1,)), ((), ()))
    dots = lax.dot_general(f, p_ref[...], dn,
                           preferred_element_type=jnp.float32)
    dotg = jnp.sum(jnp.where(oh, dots, 0.0), axis=1, keepdims=True)
    pn2g = jnp.sum(jnp.where(oh, pn2_ref[...], 0.0), axis=1, keepdims=True)
    keepg = jnp.sum(jnp.where(oh, keep_ref[...], 0.0), axis=1, keepdims=True)

    s = jnp.maximum(jnp.sqrt(sq), 1e-12)
    fn2 = sq / (s * s)
    tight = fn2 + pn2g - 2.0 * dotg / s
    sel = (gt > 0) & (keepg > 0.5)
    tmask = sel & (tight > 0.0)
    ordinal = 2.0 - gt.astype(jnp.float32) + _MARGIN
    tv = jnp.sqrt(jnp.maximum(tight, 0.0)) * ordinal
    ts_ref[...] += jnp.full((1, 1), 0.0) + jnp.sum(jnp.where(tmask, tv, 0.0))
    tc_ref[...] += jnp.full((1, 1), 0.0) + jnp.sum(
        jnp.where(tmask, 1.0, 0.0))

    @pl.when(i == n - 1)
    def _():
        tight_m = ts_ref[0, 0] / tc_ref[0, 0]
        loss_ref[...] = jnp.full((1, 1), 0.0) + (
            _LAMBDA_T_PHN * tight_m - _LAMBDA_D_PHN * ent_ref[0, 0])
        clap_out_ref[...] = clap_in_ref[...]


def kernel(features, features_text, gt, phn_id):
    N = features.shape[0] * features.shape[1]
    B = 2048
    nblk = N // B
    f32 = jnp.float32

    fs = features.reshape(N, _F)
    fts = features_text.reshape(N, _F)
    gtf = gt.reshape(N, 1).astype(jnp.int32)
    phnf = phn_id.reshape(N, 1).astype(jnp.int32)

    keep = pl.pallas_call(
        _keep_kernel,
        out_shape=jax.ShapeDtypeStruct((1, _P), f32),
    )(gtf, phnf)

    tok_spec = pl.BlockSpec((B, 1), lambda i: (i, 0))
    feat_spec = pl.BlockSpec((B, _F), lambda i: (i, 0))
    keep_spec = pl.BlockSpec((1, _P), lambda i: (0, 0))
    acc_spec = pl.BlockSpec((_P, _F), lambda i: (0, 0))
    one_spec = pl.BlockSpec((1, 1), lambda i: (0, 0))

    sums, sums_t, cnt = pl.pallas_call(
        _sums_kernel,
        grid=(nblk,),
        in_specs=[tok_spec, tok_spec, keep_spec, feat_spec, feat_spec],
        out_specs=[acc_spec, acc_spec, keep_spec],
        out_shape=[
            jax.ShapeDtypeStruct((_P, _F), f32),
            jax.ShapeDtypeStruct((_P, _F), f32),
            jax.ShapeDtypeStruct((1, _P), f32),
        ],
    )(gtf, phnf, keep, fs, fts)

    p, pn2, clap, ent = pl.pallas_call(
        _centers_kernel,
        out_shape=[
            jax.ShapeDtypeStruct((_P, _F), f32),
            jax.ShapeDtypeStruct((1, _P), f32),
            jax.ShapeDtypeStruct((1, 1), f32),
            jax.ShapeDtypeStruct((1, 1), f32),
        ],
    )(sums, sums_t, cnt)

    _, _, loss_oe, clap_out = pl.pallas_call(
        _tight_kernel,
        grid=(nblk,),
        in_specs=[tok_spec, tok_spec, keep_spec, feat_spec, acc_spec,
                  keep_spec, one_spec, one_spec],
        out_specs=[one_spec, one_spec, one_spec, one_spec],
        out_shape=[
            jax.ShapeDtypeStruct((1, 1), f32),
            jax.ShapeDtypeStruct((1, 1), f32),
            jax.ShapeDtypeStruct((1, 1), f32),
            jax.ShapeDtypeStruct((1, 1), f32),
        ],
    )(gtf, phnf, keep, fs, p, pn2, ent, clap)

    return loss_oe.reshape(()), clap_out.reshape(())


# speedup vs baseline: 5.8053x; 1.3021x over previous
import jax
import jax.numpy as jnp
from jax import lax
from jax.experimental import pallas as pl
from jax.experimental.pallas import tpu as pltpu

_LAMBDA_D_PHN = 0.1
_LAMBDA_T_PHN = 1.0
_MARGIN = 0.2
_P = 40
_F = 256
_B = 2048
_NBLK = 16


def _fused_kernel(gt_ref, phn_ref, f_ref, ft_ref,
                  loss_ref, clap_ref,
                  fbuf, sums_s, sums_t_s, cnt_n, cnt_h,
                  keep_s, p_s, pn2_s, ent_s, ts_s, tc_s):
    i = pl.program_id(0)
    iota = lax.broadcasted_iota(jnp.int32, (1, _P), 1)

    @pl.when(i == 0)
    def _():
        sums_s[...] = jnp.zeros_like(sums_s)
        sums_t_s[...] = jnp.zeros_like(sums_t_s)
        cnt_n[...] = jnp.zeros_like(cnt_n)
        cnt_h[...] = jnp.zeros_like(cnt_h)
        ts_s[...] = jnp.zeros_like(ts_s)
        tc_s[...] = jnp.zeros_like(tc_s)

    @pl.when(i < _NBLK)
    def _():
        phn = phn_ref[...]
        gt = gt_ref[...]
        f = f_ref[...]
        oh = phn == iota
        ohf = jnp.where(oh & (gt > 0), 1.0, 0.0)
        ohh = jnp.where(oh & (gt == 2), 1.0, 0.0)
        dn = (((0,), (0,)), ((), ()))
        sums_s[...] += lax.dot_general(ohf, f, dn,
                                       preferred_element_type=jnp.float32)
        sums_t_s[...] += lax.dot_general(ohf, ft_ref[...], dn,
                                         preferred_element_type=jnp.float32)
        cnt_n[...] += jnp.sum(ohf, axis=0, keepdims=True)
        cnt_h[...] += jnp.sum(ohh, axis=0, keepdims=True)
        fbuf[pl.ds(i * _B, _B), :] = f

    @pl.when(i == _NBLK)
    def _():
        r = lax.broadcasted_iota(jnp.int32, (_P, _P), 0)
        c = lax.broadcasted_iota(jnp.int32, (_P, _P), 1)
        eye = jnp.where(r == c, 1.0, 0.0)

        def row_to_col(u):
            return lax.dot_general(eye, u, (((1,), (1,)), ((), ())),
                                   preferred_element_type=jnp.float32)

        def col_to_row(v):
            return lax.dot_general(v, eye, (((0,), (0,)), ((), ())),
                                   preferred_element_type=jnp.float32)

        cn = cnt_n[...]
        ch = cnt_h[...]
        present_norm = cn > 0.0
        skip = present_norm & (~(ch > 0.0))
        any_skip = jnp.sum(jnp.where(skip, 1.0, 0.0)) > 0.0
        has_nonskip = jnp.sum(
            jnp.where(present_norm & (~skip), 1.0, 0.0)) > 0.0
        keep_if_skip = jnp.where(
            ~((iota == 1) | ((iota == 0) & has_nonskip)), 1.0, 0.0)
        keep = jnp.where(any_skip, keep_if_skip, 1.0)
        keep_s[...] = keep

        counts_raw = keep * cn
        present = counts_raw > 0.0
        counts = jnp.where(present, counts_raw, 1.0)
        counts_c = row_to_col(counts)
        keep_c = row_to_col(keep)
        n_u = jnp.sum(jnp.where(present, 1.0, 0.0))

        def norm_rows(x):
            n = jnp.sqrt(jnp.sum(x * x, axis=1, keepdims=True))
            return x / jnp.maximum(n, 1e-12)

        center = norm_rows(keep_c * sums_s[...] / counts_c)
        center_t = norm_rows(keep_c * sums_t_s[...] / counts_c)

        dn = (((1,), (1,)), ((), ()))
        logits = lax.dot_general(center, center_t, dn,
                                 preferred_element_type=jnp.float32)
        logits = jnp.where(present, logits, jnp.float32(-jnp.inf))
        m = jnp.max(logits, axis=1, keepdims=True)
        lse = jnp.log(jnp.sum(jnp.exp(logits - m), axis=1,
                              keepdims=True)) + m
        cos = logits - lse
        diag = jnp.sum(jnp.where(r == c, cos, 0.0), axis=0, keepdims=True)
        loss_a = -jnp.sum(jnp.where(present, diag, 0.0)) / n_u
        clap_ref[...] = jnp.full((1, 1), 0.0) + loss_a

        p = norm_rows(center)
        xx = jnp.sum(p * p, axis=1, keepdims=True)
        xx_r = col_to_row(xx)
        gram = lax.dot_general(p, p, dn,
                               preferred_element_type=jnp.float32)
        dist = jnp.sqrt(jnp.maximum(xx + xx_r - 2.0 * gram, 1e-12))
        present_c = row_to_col(jnp.where(present, 1.0, 0.0)) > 0.0
        pair_mask = present_c & present & (c > r)
        n_pairs = n_u * (n_u - 1.0) / 2.0
        ent_s[...] = jnp.full((1, 1), 0.0) + (
            jnp.sum(jnp.where(pair_mask, dist, 0.0)) / n_pairs)
        p_s[...] = p
        pn2_s[...] = xx_r

    @pl.when(i > _NBLK)
    def _():
        j = i - _NBLK - 1
        phn = phn_ref[...]
        gt = gt_ref[...]
        f = fbuf[pl.ds(j * _B, _B), :]
        oh = phn == iota

        sq = jnp.sum(f * f, axis=1, keepdims=True)
        dn = (((1,), (1,)), ((), ()))
        dots = lax.dot_general(f, p_s[...], dn,
                               preferred_element_type=jnp.float32)
        dotg = jnp.sum(jnp.where(oh, dots, 0.0), axis=1, keepdims=True)
        pn2g = jnp.sum(jnp.where(oh, pn2_s[...], 0.0), axis=1,
                       keepdims=True)
        keepg = jnp.sum(jnp.where(oh, keep_s[...], 0.0), axis=1,
                        keepdims=True)

        s = jnp.maximum(jnp.sqrt(sq), 1e-12)
        fn2 = sq / (s * s)
        tight = fn2 + pn2g - 2.0 * dotg / s
        tmask = (gt > 0) & (keepg > 0.5) & (tight > 0.0)
        ordinal = 2.0 - gt.astype(jnp.float32) + _MARGIN
        tv = jnp.sqrt(jnp.maximum(tight, 0.0)) * ordinal
        ts_s[...] += jnp.full((1, 1), 0.0) + jnp.sum(
            jnp.where(tmask, tv, 0.0))
        tc_s[...] += jnp.full((1, 1), 0.0) + jnp.sum(
            jnp.where(tmask, 1.0, 0.0))

        @pl.when(i == 2 * _NBLK)
        def _():
            tight_m = ts_s[0, 0] / tc_s[0, 0]
            loss_ref[...] = jnp.full((1, 1), 0.0) + (
                _LAMBDA_T_PHN * tight_m - _LAMBDA_D_PHN * ent_s[0, 0])


def kernel(features, features_text, gt, phn_id):
    N = features.shape[0] * features.shape[1]
    f32 = jnp.float32

    fs = features.reshape(N, _F)
    fts = features_text.reshape(N, _F)
    gtf = gt.reshape(N, 1).astype(jnp.int32)
    phnf = phn_id.reshape(N, 1).astype(jnp.int32)

    def tok_map(i):
        j = jnp.where(i < _NBLK, i,
                      jnp.where(i == _NBLK, _NBLK - 1, i - _NBLK - 1))
        return (j, 0)

    def feat_map(i):
        return (jnp.minimum(i, _NBLK - 1), 0)

    tok_spec = pl.BlockSpec((_B, 1), tok_map)
    feat_spec = pl.BlockSpec((_B, _F), feat_map)
    one_spec = pl.BlockSpec((1, 1), lambda i: (0, 0))

    loss_oe, clap = pl.pallas_call(
        _fused_kernel,
        grid=(2 * _NBLK + 1,),
        in_specs=[tok_spec, tok_spec, feat_spec, feat_spec],
        out_specs=[one_spec, one_spec],
        out_shape=[
            jax.ShapeDtypeStruct((1, 1), f32),
            jax.ShapeDtypeStruct((1, 1), f32),
        ],
        scratch_shapes=[
            pltpu.VMEM((N, _F), f32),
            pltpu.VMEM((_P, _F), f32),
            pltpu.VMEM((_P, _F), f32),
            pltpu.VMEM((1, _P), f32),
            pltpu.VMEM((1, _P), f32),
            pltpu.VMEM((1, _P), f32),
            pltpu.VMEM((_P, _F), f32),
            pltpu.VMEM((1, _P), f32),
            pltpu.VMEM((1, 1), f32),
            pltpu.VMEM((1, 1), f32),
            pltpu.VMEM((1, 1), f32),
        ],
    )(gtf, phnf, fs, fts)

    return loss_oe.reshape(()), clap.reshape(())


# speedup vs baseline: 6.1395x; 1.0576x over previous
import jax
import jax.numpy as jnp
from jax import lax
from jax.experimental import pallas as pl
from jax.experimental.pallas import tpu as pltpu

_LAMBDA_D_PHN = 0.1
_LAMBDA_T_PHN = 1.0
_MARGIN = 0.2
_P = 40
_F = 256
_B = 2048
_NBLK = 16


def _fused_kernel(gt_ref, phn_ref, gtr_ref, phnr_ref, f_ref, ft_ref,
                  loss_ref, clap_ref,
                  fbuf, sums_s, sums_t_s, cnt2_s,
                  p_s, k_s, ts_s, tc_s):
    i = pl.program_id(0)

    @pl.when(i == 0)
    def _():
        sums_s[...] = jnp.zeros_like(sums_s)
        sums_t_s[...] = jnp.zeros_like(sums_t_s)
        cnt2_s[...] = jnp.zeros_like(cnt2_s)
        ts_s[0] = 0.0
        tc_s[0] = 0.0

    @pl.when(i < _NBLK)
    def _():
        phn_r = phnr_ref[0]
        gt_r = gtr_ref[0]
        gt_c = gt_ref[...]
        f = f_ref[...]
        iota_c = lax.broadcasted_iota(jnp.int32, (_P, 1), 0)
        ohT = jnp.where((phn_r == iota_c) & (gt_r > 0), 1.0, 0.0)
        dn = (((1,), (0,)), ((), ()))
        sums_s[...] += lax.dot_general(ohT, f, dn,
                                       preferred_element_type=jnp.float32)
        sums_t_s[...] += lax.dot_general(ohT, ft_ref[...], dn,
                                         preferred_element_type=jnp.float32)
        ones_c = jnp.full((_B, 1), 1.0)
        hind_c = jnp.where(gt_c == 2, 1.0, 0.0)
        aug2 = jnp.concatenate([ones_c, hind_c], axis=1)
        cnt2_s[...] += lax.dot_general(ohT, aug2, dn,
                                       preferred_element_type=jnp.float32)
        fbuf[pl.ds(i * _B, _B), :] = f

    @pl.when(i == _NBLK)
    def _():
        r = lax.broadcasted_iota(jnp.int32, (_P, _P), 0)
        c = lax.broadcasted_iota(jnp.int32, (_P, _P), 1)
        eye = jnp.where(r == c, 1.0, 0.0)

        def col_to_row(v):
            return lax.dot_general(v, eye, (((0,), (0,)), ((), ())),
                                   preferred_element_type=jnp.float32)

        iota_c = lax.broadcasted_iota(jnp.int32, (_P, 1), 0)
        cn_c = cnt2_s[:, 0:1]
        ch_c = cnt2_s[:, 1:2]
        present_norm = cn_c > 0.0
        skip = present_norm & (~(ch_c > 0.0))
        any_skip = jnp.sum(jnp.where(skip, 1.0, 0.0)) > 0.0
        has_nonskip = jnp.sum(
            jnp.where(present_norm & (~skip), 1.0, 0.0)) > 0.0
        keep_if_skip = jnp.where(
            ~((iota_c == 1) | ((iota_c == 0) & has_nonskip)), 1.0, 0.0)
        keep_c = jnp.where(any_skip, keep_if_skip, 1.0)
        k_s[0] = jnp.where(any_skip & has_nonskip, 1.0, 0.0)
        k_s[1] = jnp.where(any_skip, 1.0, 0.0)

        counts_raw = keep_c * cn_c
        present_c = counts_raw > 0.0
        counts_c = jnp.where(present_c, counts_raw, 1.0)
        n_u = jnp.sum(jnp.where(present_c, 1.0, 0.0))

        def norm_rows(x):
            n = jnp.sqrt(jnp.sum(x * x, axis=1, keepdims=True))
            return x / jnp.maximum(n, 1e-12)

        center = norm_rows(keep_c * sums_s[...] / counts_c)
        center_t = norm_rows(keep_c * sums_t_s[...] / counts_c)

        dn = (((1,), (1,)), ((), ()))
        logits = lax.dot_general(center, center_t, dn,
                                 preferred_element_type=jnp.float32)
        present_r = col_to_row(jnp.where(present_c, 1.0, 0.0)) > 0.0
        logits = jnp.where(present_r, logits, jnp.float32(-jnp.inf))
        m = jnp.max(logits, axis=1, keepdims=True)
        lse = jnp.log(jnp.sum(jnp.exp(logits - m), axis=1,
                              keepdims=True)) + m
        cos = logits - lse
        diag = jnp.sum(jnp.where(r == c, cos, 0.0), axis=0, keepdims=True)
        loss_a = -jnp.sum(jnp.where(present_r, diag, 0.0)) / n_u
        clap_ref[...] = jnp.full((1, 1), 0.0) + loss_a

        p = norm_rows(center)
        xx = jnp.sum(p * p, axis=1, keepdims=True)
        xx_r = col_to_row(xx)
        gram = lax.dot_general(p, p, dn,
                               preferred_element_type=jnp.float32)
        dist = jnp.sqrt(jnp.maximum(xx + xx_r - 2.0 * gram, 1e-12))
        pair_mask = present_c & present_r & (c > r)
        n_pairs = n_u * (n_u - 1.0) / 2.0
        k_s[2] = jnp.sum(jnp.where(pair_mask, dist, 0.0)) / n_pairs
        p_s[...] = p

    @pl.when(i > _NBLK)
    def _():
        j = i - _NBLK - 1
        phn = phn_ref[...]
        gt = gt_ref[...]
        f = fbuf[pl.ds(j * _B, _B), :]
        iota = lax.broadcasted_iota(jnp.int32, (1, _P), 1)
        oh = phn == iota

        sq = jnp.sum(f * f, axis=1, keepdims=True)
        dn = (((1,), (1,)), ((), ()))
        dots = lax.dot_general(f, p_s[...], dn,
                               preferred_element_type=jnp.float32)
        dotg = jnp.sum(jnp.where(oh, dots, 0.0), axis=1, keepdims=True)

        s = jnp.maximum(jnp.sqrt(sq), 1e-12)
        tight = sq / (s * s) + 1.0 - 2.0 * dotg / s
        killed = (jnp.where(phn == 0, k_s[0], 0.0)
                  + jnp.where(phn == 1, k_s[1], 0.0))
        tmask = (gt > 0) & (killed < 0.5) & (tight > 0.0)
        ordinal = 2.0 - gt.astype(jnp.float32) + _MARGIN
        tv = jnp.sqrt(jnp.maximum(tight, 0.0)) * ordinal
        ts_s[0] += jnp.sum(jnp.where(tmask, tv, 0.0))
        tc_s[0] += jnp.sum(jnp.where(tmask, 1.0, 0.0))

        @pl.when(i == 2 * _NBLK)
        def _():
            tight_m = ts_s[0] / tc_s[0]
            loss_ref[...] = jnp.full((1, 1), 0.0) + (
                _LAMBDA_T_PHN * tight_m - _LAMBDA_D_PHN * k_s[2])


def kernel(features, features_text, gt, phn_id):
    N = features.shape[0] * features.shape[1]
    f32 = jnp.float32

    fs = features.reshape(N, _F)
    fts = features_text.reshape(N, _F)
    gtf = gt.reshape(N, 1).astype(jnp.int32)
    phnf = phn_id.reshape(N, 1).astype(jnp.int32)
    gtr = gt.reshape(_NBLK, 1, _B).astype(jnp.int32)
    phnr = phn_id.reshape(_NBLK, 1, _B).astype(jnp.int32)

    def tok_map(i):
        j = jnp.where(i < _NBLK, i,
                      jnp.where(i == _NBLK, _NBLK - 1, i - _NBLK - 1))
        return (j, 0)

    def tok3_map(i):
        return (jnp.minimum(i, _NBLK - 1), 0, 0)

    def feat_map(i):
        return (jnp.minimum(i, _NBLK - 1), 0)

    tok_spec = pl.BlockSpec((_B, 1), tok_map)
    tok3_spec = pl.BlockSpec((1, 1, _B), tok3_map)
    feat_spec = pl.BlockSpec((_B, _F), feat_map)
    one_spec = pl.BlockSpec((1, 1), lambda i: (0, 0))

    loss_oe, clap = pl.pallas_call(
        _fused_kernel,
        grid=(2 * _NBLK + 1,),
        in_specs=[tok_spec, tok_spec, tok3_spec, tok3_spec,
                  feat_spec, feat_spec],
        out_specs=[one_spec, one_spec],
        out_shape=[
            jax.ShapeDtypeStruct((1, 1), f32),
            jax.ShapeDtypeStruct((1, 1), f32),
        ],
        scratch_shapes=[
            pltpu.VMEM((N, _F), f32),
            pltpu.VMEM((_P, _F), f32),
            pltpu.VMEM((_P, _F), f32),
            pltpu.VMEM((_P, 2), f32),
            pltpu.VMEM((_P, _F), f32),
            pltpu.SMEM((3,), f32),
            pltpu.SMEM((1,), f32),
            pltpu.SMEM((1,), f32),
        ],
    )(gtf, phnf, gtr, phnr, fs, fts)

    return loss_oe.reshape(()), clap.reshape(())


# speedup vs baseline: 14.0353x; 2.2860x over previous
import jax
import jax.numpy as jnp
from jax import lax
from jax.experimental import pallas as pl
from jax.experimental.pallas import tpu as pltpu

_LAMBDA_D_PHN = 0.1
_LAMBDA_T_PHN = 1.0
_MARGIN = 0.2
_P = 40
_F = 256
_B = 4096
_NBLK = 8


def _fused_kernel(gtr_ref, phnr_ref, f_ref, ft_ref,
                  loss_ref, clap_ref,
                  fbuf, sums_s, sums_t_s, cnt2_s,
                  p_s, k_s, ts_s, tc_s):
    i = pl.program_id(0)

    @pl.when(i == 0)
    def _():
        sums_s[...] = jnp.zeros_like(sums_s)
        sums_t_s[...] = jnp.zeros_like(sums_t_s)
        cnt2_s[...] = jnp.zeros_like(cnt2_s)
        ts_s[0] = 0.0
        tc_s[0] = 0.0

    @pl.when(i < _NBLK)
    def _():
        phn_r = phnr_ref[0]
        gt_r = gtr_ref[0]
        f = f_ref[...]
        iota_c = lax.broadcasted_iota(jnp.int32, (_P, 1), 0)
        ohT = jnp.where((phn_r == iota_c) & (gt_r > 0), 1.0, 0.0)
        dn = (((1,), (0,)), ((), ()))
        sums_s[...] += lax.dot_general(ohT, f, dn,
                                       preferred_element_type=jnp.float32)
        sums_t_s[...] += lax.dot_general(ohT, ft_ref[...], dn,
                                         preferred_element_type=jnp.float32)
        hind_r = jnp.where(gt_r == 2, 1.0, 0.0)
        cn = jnp.sum(ohT, axis=1, keepdims=True)
        ch = jnp.sum(ohT * hind_r, axis=1, keepdims=True)
        cnt2_s[...] += jnp.concatenate([cn, ch], axis=1)
        fbuf[pl.ds(i * _B, _B), :] = f

    @pl.when(i == _NBLK)
    def _():
        r = lax.broadcasted_iota(jnp.int32, (_P, _P), 0)
        c = lax.broadcasted_iota(jnp.int32, (_P, _P), 1)
        eye = jnp.where(r == c, 1.0, 0.0)

        def col_to_row(v):
            return lax.dot_general(v, eye, (((0,), (0,)), ((), ())),
                                   preferred_element_type=jnp.float32)

        iota_c = lax.broadcasted_iota(jnp.int32, (_P, 1), 0)
        cn_c = cnt2_s[:, 0:1]
        ch_c = cnt2_s[:, 1:2]
        present_norm = cn_c > 0.0
        skip = present_norm & (~(ch_c > 0.0))
        any_skip = jnp.sum(jnp.where(skip, 1.0, 0.0)) > 0.0
        has_nonskip = jnp.sum(
            jnp.where(present_norm & (~skip), 1.0, 0.0)) > 0.0
        keep_if_skip = jnp.where(
            ~((iota_c == 1) | ((iota_c == 0) & has_nonskip)), 1.0, 0.0)
        keep_c = jnp.where(any_skip, keep_if_skip, 1.0)
        k_s[0] = jnp.where(any_skip & has_nonskip, 1.0, 0.0)
        k_s[1] = jnp.where(any_skip, 1.0, 0.0)

        counts_raw = keep_c * cn_c
        present_c = counts_raw > 0.0
        counts_c = jnp.where(present_c, counts_raw, 1.0)
        n_u = jnp.sum(jnp.where(present_c, 1.0, 0.0))

        def norm_rows(x):
            n = jnp.sqrt(jnp.sum(x * x, axis=1, keepdims=True))
            return x / jnp.maximum(n, 1e-12)

        center = norm_rows(keep_c * sums_s[...] / counts_c)
        center_t = norm_rows(keep_c * sums_t_s[...] / counts_c)

        dn = (((1,), (1,)), ((), ()))
        logits = lax.dot_general(center, center_t, dn,
                                 preferred_element_type=jnp.float32)
        present_r = col_to_row(jnp.where(present_c, 1.0, 0.0)) > 0.0
        logits = jnp.where(present_r, logits, jnp.float32(-jnp.inf))
        m = jnp.max(logits, axis=1, keepdims=True)
        lse = jnp.log(jnp.sum(jnp.exp(logits - m), axis=1,
                              keepdims=True)) + m
        cos = logits - lse
        diag = jnp.sum(jnp.where(r == c, cos, 0.0), axis=0, keepdims=True)
        loss_a = -jnp.sum(jnp.where(present_r, diag, 0.0)) / n_u
        clap_ref[...] = jnp.full((1, 1), 0.0) + loss_a

        p = norm_rows(center)
        xx = jnp.sum(p * p, axis=1, keepdims=True)
        xx_r = col_to_row(xx)
        gram = lax.dot_general(p, p, dn,
                               preferred_element_type=jnp.float32)
        dist = jnp.sqrt(jnp.maximum(xx + xx_r - 2.0 * gram, 1e-12))
        pair_mask = present_c & present_r & (c > r)
        n_pairs = n_u * (n_u - 1.0) / 2.0
        k_s[2] = jnp.sum(jnp.where(pair_mask, dist, 0.0)) / n_pairs
        p_s[...] = p

    @pl.when(i > _NBLK)
    def _():
        j = i - _NBLK - 1
        phn_r = phnr_ref[0]
        gt_r = gtr_ref[0]
        f = fbuf[pl.ds(j * _B, _B), :]
        iota_c = lax.broadcasted_iota(jnp.int32, (_P, 1), 0)
        ohT = phn_r == iota_c

        ones_f = jnp.full((1, _F), 1.0)
        dn = (((1,), (1,)), ((), ()))
        sq = lax.dot_general(ones_f, f * f, dn,
                             preferred_element_type=jnp.float32)
        dotsT = lax.dot_general(p_s[...], f, dn,
                                preferred_element_type=jnp.float32)
        dotg = jnp.sum(jnp.where(ohT, dotsT, 0.0), axis=0,
                       keepdims=True)

        s = jnp.maximum(jnp.sqrt(sq), 1e-12)
        tight = sq / (s * s) + 1.0 - 2.0 * dotg / s
        killed = (jnp.where(phn_r == 0, k_s[0], 0.0)
                  + jnp.where(phn_r == 1, k_s[1], 0.0))
        tmask = (gt_r > 0) & (killed < 0.5) & (tight > 0.0)
        ordinal = 2.0 - gt_r.astype(jnp.float32) + _MARGIN
        tv = jnp.sqrt(jnp.maximum(tight, 0.0)) * ordinal
        ts_s[0] += jnp.sum(jnp.where(tmask, tv, 0.0))
        tc_s[0] += jnp.sum(jnp.where(tmask, 1.0, 0.0))

        @pl.when(i == 2 * _NBLK)
        def _():
            tight_m = ts_s[0] / tc_s[0]
            loss_ref[...] = jnp.full((1, 1), 0.0) + (
                _LAMBDA_T_PHN * tight_m - _LAMBDA_D_PHN * k_s[2])


def kernel(features, features_text, gt, phn_id):
    N = features.shape[0] * features.shape[1]
    f32 = jnp.float32

    fs = features.reshape(N, _F)
    fts = features_text.reshape(N, _F)
    gtr = gt.reshape(_NBLK, 1, _B).astype(jnp.int32)
    phnr = phn_id.reshape(_NBLK, 1, _B).astype(jnp.int32)

    def tok3_map(i):
        j = jnp.where(i < _NBLK, i,
                      jnp.where(i == _NBLK, _NBLK - 1, i - _NBLK - 1))
        return (j, 0, 0)

    def feat_map(i):
        return (jnp.minimum(i, _NBLK - 1), 0)

    tok3_spec = pl.BlockSpec((1, 1, _B), tok3_map)
    feat_spec = pl.BlockSpec((_B, _F), feat_map)
    one_spec = pl.BlockSpec((1, 1), lambda i: (0, 0))

    loss_oe, clap = pl.pallas_call(
        _fused_kernel,
        grid=(2 * _NBLK + 1,),
        in_specs=[tok3_spec, tok3_spec, feat_spec, feat_spec],
        out_specs=[one_spec, one_spec],
        out_shape=[
            jax.ShapeDtypeStruct((1, 1), f32),
            jax.ShapeDtypeStruct((1, 1), f32),
        ],
        scratch_shapes=[
            pltpu.VMEM((N, _F), f32),
            pltpu.VMEM((_P, _F), f32),
            pltpu.VMEM((_P, _F), f32),
            pltpu.VMEM((_P, 2), f32),
            pltpu.VMEM((_P, _F), f32),
            pltpu.SMEM((3,), f32),
            pltpu.SMEM((1,), f32),
            pltpu.SMEM((1,), f32),
        ],
    )(gtr, phnr, fs, fts)

    return loss_oe.reshape(()), clap.reshape(())
